# baseline (speedup 1.0000x reference)
import jax
import jax.numpy as jnp
from jax import lax
from jax.experimental import pallas as pl
from jax.experimental.pallas import tpu as pltpu

TEMP = 0.1
BASE_TEMP = 1.0
FLAT = 120000
KT = 3840
NSTEPS = (FLAT + KT - 1) // KT
PAR = 2
HALF = NSTEPS // PAR
NEG = -1e30

_INTERPRET = False


def _mm_kernel(xc_ref, xf1_ref, xf2_ref, w1_ref, outc_ref, outf_ref,
               acc_c, acc_f1, acc_f2):
    m = pl.program_id(0)
    k = pl.program_id(1)
    g = m * HALF + k

    @pl.when(k == 0)
    def _init():
        acc_c[...] = jnp.zeros_like(acc_c)
        acc_f1[...] = jnp.zeros_like(acc_f1)
        acc_f2[...] = jnp.zeros_like(acc_f2)

    @pl.when(g < NSTEPS - 1)
    def _full():
        w1 = w1_ref[...]
        acc_c[...] += jnp.dot(xc_ref[...], w1, preferred_element_type=jnp.float32)
        acc_f1[...] += jnp.dot(xf1_ref[...], w1, preferred_element_type=jnp.float32)
        acc_f2[...] += jnp.dot(xf2_ref[...], w1, preferred_element_type=jnp.float32)

    @pl.when(g == NSTEPS - 1)
    def _last():
        rem = FLAT - (NSTEPS - 1) * KT
        colmask = lax.broadcasted_iota(jnp.int32, (1, KT), 1) < rem
        rowmask = lax.broadcasted_iota(jnp.int32, (KT, 1), 0) < rem
        xc = jnp.where(colmask, xc_ref[...], 0.0)
        xf1 = jnp.where(colmask, xf1_ref[...], 0.0)
        xf2 = jnp.where(colmask, xf2_ref[...], 0.0)
        w1 = jnp.where(rowmask, w1_ref[...], 0.0)
        acc_c[...] += jnp.dot(xc, w1, preferred_element_type=jnp.float32)
        acc_f1[...] += jnp.dot(xf1, w1, preferred_element_type=jnp.float32)
        acc_f2[...] += jnp.dot(xf2, w1, preferred_element_type=jnp.float32)

    @pl.when(k == HALF - 1)
    def _flush():
        outc_ref[...] = acc_c[...][None]
        outf_ref[0, 0:128, :] = acc_f1[...]
        outf_ref[0, 128:256, :] = acc_f2[...]


def _loss_kernel(pc_ref, pf_ref,
                 lab_row_ref, lab_col_ref, ic_row_ref, ic_col_ref, iff_row_ref,
                 t1_row_ref, t2_row_ref, t1_col_ref, t2_col_ref, clsf_row_ref,
                 b1_ref, b2_ref, w2_ref, out_ref):
    b1 = b1_ref[...]
    b2 = b2_ref[...]
    w2 = w2_ref[...]
    hc = jnp.maximum(pc_ref[0] + pc_ref[1] + b1, 0.0)
    hf = jnp.maximum(pf_ref[0] + pf_ref[1] + b1, 0.0)
    aemb = jnp.dot(hc, w2, preferred_element_type=jnp.float32) + b2
    femb = jnp.dot(hf, w2, preferred_element_type=jnp.float32) + b2
    zemb = jnp.dot(jnp.maximum(b1, 0.0), w2,
                   preferred_element_type=jnp.float32) + b2

    lab_row = lab_row_ref[...]
    lab_col = lab_col_ref[...]
    ic_row = ic_row_ref[...]
    ic_col = ic_col_ref[...]
    iff_row = iff_row_ref[...]
    t1_row = t1_row_ref[...]
    t2_row = t2_row_ref[...]
    t1_col = t1_col_ref[...]
    t2_col = t2_col_ref[...]
    clsf_row = clsf_row_ref[...]

    oh_c = (lax.broadcasted_iota(jnp.int32, (512, 128), 0)
            == ic_row).astype(jnp.float32)
    lab_ic_row = jnp.dot(lab_row, oh_c, preferred_element_type=jnp.float32)
    oh_ct = (ic_col == lax.broadcasted_iota(jnp.int32, (128, 512), 1)
             ).astype(jnp.float32)
    lab_ic_col = jnp.dot(oh_ct, lab_col, preferred_element_type=jnp.float32)
    oh_f = (lax.broadcasted_iota(jnp.int32, (512, 256), 0)
            == iff_row).astype(jnp.float32)
    lab_iff_row = jnp.dot(lab_row, oh_f, preferred_element_type=jnp.float32)

    cond_col = (t1_col != lab_ic_col) & (t2_col == lab_ic_col)
    cond_row = (t1_row != lab_ic_row) & (t2_row == lab_ic_row)
    condf_row = clsf_row == lab_iff_row
    n_sel = jnp.sum(cond_col.astype(jnp.float32))

    femb2 = femb * femb
    aemb2 = aemb * aemb
    aemb_m = jnp.where(cond_col, aemb, 0.0)
    colnorm = jnp.sqrt(jnp.sum(aemb_m * aemb_m, axis=0, keepdims=True))
    anchor = aemb_m / jnp.maximum(colnorm, 1e-12)
    z2 = zemb * zemb

    posmask = condf_row & (clsf_row == t2_col)
    pm = posmask.astype(jnp.float32)
    lenP = jnp.sum(pm, axis=1, keepdims=True)
    maxP = jnp.max(jnp.where(cond_col, lenP, 0.0))
    norm2P = (jnp.dot(pm, femb2, preferred_element_type=jnp.float32)
              + (maxP - lenP) * z2)
    denP = jnp.maximum(jnp.sqrt(norm2P), 1e-12)
    anchorP = anchor / denP
    num = lax.dot_general(anchorP, femb, (((1,), (1,)), ((), ())),
                          preferred_element_type=jnp.float32) / TEMP
    pad_num = jnp.sum(anchorP * zemb, axis=1, keepdims=True) / TEMP

    maskA = condf_row & (clsf_row == t1_col)
    maskB = cond_row & (t1_row == t2_col)
    ma = maskA.astype(jnp.float32)
    mb = maskB.astype(jnp.float32)
    lenQ = (jnp.sum(ma, axis=1, keepdims=True)
            + jnp.sum(mb, axis=1, keepdims=True))
    maxQ = jnp.max(jnp.where(cond_col, lenQ, 0.0))
    norm2Q = (jnp.dot(ma, femb2, preferred_element_type=jnp.float32)
              + jnp.dot(mb, aemb2, preferred_element_type=jnp.float32)
              + (maxQ - lenQ) * z2)
    denQ = jnp.maximum(jnp.sqrt(norm2Q), 1e-12)
    anchorQ = anchor / denQ
    sF = lax.dot_general(anchorQ, femb, (((1,), (1,)), ((), ())),
                         preferred_element_type=jnp.float32) / TEMP
    sB = lax.dot_general(anchorQ, aemb, (((1,), (1,)), ((), ())),
                         preferred_element_type=jnp.float32) / TEMP
    sFm = jnp.where(maskA, sF, NEG)
    sBm = jnp.where(maskB, sB, NEG)
    has_pad = lenQ < maxQ
    m = jnp.maximum(jnp.max(sFm, axis=1, keepdims=True),
                    jnp.max(sBm, axis=1, keepdims=True))
    m = jnp.maximum(m, jnp.where(has_pad, 0.0, NEG))
    expF = jnp.where(maskA, jnp.exp(sFm - m), 0.0)
    expB = jnp.where(maskB, jnp.exp(sBm - m), 0.0)
    logsum = jnp.log(jnp.sum(expF, axis=1, keepdims=True)
                     + jnp.sum(expB, axis=1, keepdims=True)
                     + (maxQ - lenQ) * jnp.exp(-m))
    numsum = (jnp.sum(jnp.where(posmask, num, 0.0), axis=1, keepdims=True)
              + (maxP - lenP) * pad_num)
    mean_lp = (numsum - maxP * logsum) / maxP
    loss = -(TEMP / BASE_TEMP) * mean_lp
    total = jnp.sum(jnp.where(cond_col, loss, 0.0)) / n_sel
    out_ref[...] = jnp.full((1, 128), total, jnp.float32)


def kernel(label, samples_of_further_pairs, class_of_further_pair,
           idx_further_pair, samples_of_closest_pairs, class_of_closest_pair,
           idx_closest_pair, W1, b1, W2, b2):
    Xc = samples_of_closest_pairs.reshape(128, FLAT)
    Xf = samples_of_further_pairs.reshape(256, FLAT)
    labf = label.astype(jnp.float32)
    lab_row = labf.reshape(1, 512)
    lab_col = labf.reshape(512, 1)
    ic = idx_closest_pair.astype(jnp.int32)
    ic_row = ic.reshape(1, 128)
    ic_col = ic.reshape(128, 1)
    iff_row = idx_further_pair.astype(jnp.int32).reshape(1, 256)
    ccpf = class_of_closest_pair.astype(jnp.float32)
    t1_row = ccpf[:, 0].reshape(1, 128)
    t2_row = ccpf[:, 1].reshape(1, 128)
    t1_col = ccpf[:, 0].reshape(128, 1)
    t2_col = ccpf[:, 1].reshape(128, 1)
    clsf_row = class_of_further_pair[:, 0].astype(jnp.float32).reshape(1, 256)
    b1r = b1.reshape(1, 128)
    b2r = b2.reshape(1, 128)

    pc, pf = pl.pallas_call(
        _mm_kernel,
        grid=(PAR, HALF),
        in_specs=[
            pl.BlockSpec((128, KT), lambda m, k: (0, m * HALF + k)),
            pl.BlockSpec((128, KT), lambda m, k: (0, m * HALF + k)),
            pl.BlockSpec((128, KT), lambda m, k: (1, m * HALF + k)),
            pl.BlockSpec((KT, 128), lambda m, k: (m * HALF + k, 0)),
        ],
        out_specs=[
            pl.BlockSpec((1, 128, 128), lambda m, k: (m, 0, 0)),
            pl.BlockSpec((1, 256, 128), lambda m, k: (m, 0, 0)),
        ],
        out_shape=[jax.ShapeDtypeStruct((PAR, 128, 128), jnp.float32),
                   jax.ShapeDtypeStruct((PAR, 256, 128), jnp.float32)],
        scratch_shapes=[pltpu.VMEM((128, 128), jnp.float32),
                        pltpu.VMEM((128, 128), jnp.float32),
                        pltpu.VMEM((128, 128), jnp.float32)],
        compiler_params=pltpu.CompilerParams(
            dimension_semantics=("parallel", "arbitrary")),
        interpret=_INTERPRET,
    )(Xc, Xf, Xf, W1)

    def full(shape):
        return pl.BlockSpec(shape, lambda: (0,) * len(shape))

    out = pl.pallas_call(
        _loss_kernel,
        in_specs=[
            full((PAR, 128, 128)), full((PAR, 256, 128)),
            full((1, 512)), full((512, 1)), full((1, 128)), full((128, 1)),
            full((1, 256)), full((1, 128)), full((1, 128)), full((128, 1)),
            full((128, 1)), full((1, 256)), full((1, 128)), full((1, 128)),
            full((128, 128)),
        ],
        out_specs=full((1, 128)),
        out_shape=jax.ShapeDtypeStruct((1, 128), jnp.float32),
        interpret=_INTERPRET,
    )(pc, pf, lab_row, lab_col, ic_row, ic_col, iff_row,
      t1_row, t2_row, t1_col, t2_col, clsf_row, b1r, b2r, W2)
    return out[0, 0]


# speedup vs baseline: 1.0013x; 1.0013x over previous
import jax
import jax.numpy as jnp
from jax import lax
from jax.experimental import pallas as pl
from jax.experimental.pallas import tpu as pltpu

TEMP = 0.1
BASE_TEMP = 1.0
FLAT = 120000
KT = 3840
NSTEPS = (FLAT + KT - 1) // KT
PAR = 2
HALF = NSTEPS // PAR
NEG = -1e30

_INTERPRET = False


def _mm_kernel(xc_ref, xf_ref, w1_ref, outc_ref, outf_ref, acc_c, acc_f):
    m = pl.program_id(0)
    k = pl.program_id(1)
    g = m * HALF + k

    @pl.when(k == 0)
    def _init():
        acc_c[...] = jnp.zeros_like(acc_c)
        acc_f[...] = jnp.zeros_like(acc_f)

    @pl.when(g < NSTEPS - 1)
    def _full():
        w1 = w1_ref[...]
        acc_c[...] += jnp.dot(xc_ref[...], w1, preferred_element_type=jnp.float32)
        acc_f[...] += jnp.dot(xf_ref[...], w1, preferred_element_type=jnp.float32)

    @pl.when(g == NSTEPS - 1)
    def _last():
        rem = FLAT - (NSTEPS - 1) * KT
        colmask = lax.broadcasted_iota(jnp.int32, (1, KT), 1) < rem
        rowmask = lax.broadcasted_iota(jnp.int32, (KT, 1), 0) < rem
        xc = jnp.where(colmask, xc_ref[...], 0.0)
        xf = jnp.where(colmask, xf_ref[...], 0.0)
        w1 = jnp.where(rowmask, w1_ref[...], 0.0)
        acc_c[...] += jnp.dot(xc, w1, preferred_element_type=jnp.float32)
        acc_f[...] += jnp.dot(xf, w1, preferred_element_type=jnp.float32)

    @pl.when(k == HALF - 1)
    def _flush():
        outc_ref[...] = acc_c[...][None]
        outf_ref[...] = acc_f[...][None]


def _loss_kernel(pc_ref, pf_ref,
                 lab_row_ref, lab_col_ref, ic_row_ref, ic_col_ref, iff_row_ref,
                 t1_row_ref, t2_row_ref, t1_col_ref, t2_col_ref, clsf_row_ref,
                 b1_ref, b2_ref, w2_ref, out_ref):
    b1 = b1_ref[...]
    b2 = b2_ref[...]
    w2 = w2_ref[...]
    hc = jnp.maximum(pc_ref[0] + pc_ref[1] + b1, 0.0)
    hf = jnp.maximum(pf_ref[0] + pf_ref[1] + b1, 0.0)
    aemb = jnp.dot(hc, w2, preferred_element_type=jnp.float32) + b2
    femb = jnp.dot(hf, w2, preferred_element_type=jnp.float32) + b2
    zemb = jnp.dot(jnp.maximum(b1, 0.0), w2,
                   preferred_element_type=jnp.float32) + b2

    lab_row = lab_row_ref[...]
    lab_col = lab_col_ref[...]
    ic_row = ic_row_ref[...]
    ic_col = ic_col_ref[...]
    iff_row = iff_row_ref[...]
    t1_row = t1_row_ref[...]
    t2_row = t2_row_ref[...]
    t1_col = t1_col_ref[...]
    t2_col = t2_col_ref[...]
    clsf_row = clsf_row_ref[...]

    oh_c = (lax.broadcasted_iota(jnp.int32, (512, 128), 0)
            == ic_row).astype(jnp.float32)
    lab_ic_row = jnp.dot(lab_row, oh_c, preferred_element_type=jnp.float32)
    oh_ct = (ic_col == lax.broadcasted_iota(jnp.int32, (128, 512), 1)
             ).astype(jnp.float32)
    lab_ic_col = jnp.dot(oh_ct, lab_col, preferred_element_type=jnp.float32)
    oh_f = (lax.broadcasted_iota(jnp.int32, (512, 256), 0)
            == iff_row).astype(jnp.float32)
    lab_iff_row = jnp.dot(lab_row, oh_f, preferred_element_type=jnp.float32)

    cond_col = (t1_col != lab_ic_col) & (t2_col == lab_ic_col)
    cond_row = (t1_row != lab_ic_row) & (t2_row == lab_ic_row)
    condf_row = clsf_row == lab_iff_row
    n_sel = jnp.sum(cond_col.astype(jnp.float32))

    femb2 = femb * femb
    aemb2 = aemb * aemb
    aemb_m = jnp.where(cond_col, aemb, 0.0)
    colnorm = jnp.sqrt(jnp.sum(aemb_m * aemb_m, axis=0, keepdims=True))
    anchor = aemb_m / jnp.maximum(colnorm, 1e-12)
    z2 = zemb * zemb

    posmask = condf_row & (clsf_row == t2_col)
    pm = posmask.astype(jnp.float32)
    lenP = jnp.sum(pm, axis=1, keepdims=True)
    maxP = jnp.max(jnp.where(cond_col, lenP, 0.0))
    norm2P = (jnp.dot(pm, femb2, preferred_element_type=jnp.float32)
              + (maxP - lenP) * z2)
    denP = jnp.maximum(jnp.sqrt(norm2P), 1e-12)
    anchorP = anchor / denP
    num = lax.dot_general(anchorP, femb, (((1,), (1,)), ((), ())),
                          preferred_element_type=jnp.float32) / TEMP
    pad_num = jnp.sum(anchorP * zemb, axis=1, keepdims=True) / TEMP

    maskA = condf_row & (clsf_row == t1_col)
    maskB = cond_row & (t1_row == t2_col)
    ma = maskA.astype(jnp.float32)
    mb = maskB.astype(jnp.float32)
    lenQ = (jnp.sum(ma, axis=1, keepdims=True)
            + jnp.sum(mb, axis=1, keepdims=True))
    maxQ = jnp.max(jnp.where(cond_col, lenQ, 0.0))
    norm2Q = (jnp.dot(ma, femb2, preferred_element_type=jnp.float32)
              + jnp.dot(mb, aemb2, preferred_element_type=jnp.float32)
              + (maxQ - lenQ) * z2)
    denQ = jnp.maximum(jnp.sqrt(norm2Q), 1e-12)
    anchorQ = anchor / denQ
    sF = lax.dot_general(anchorQ, femb, (((1,), (1,)), ((), ())),
                         preferred_element_type=jnp.float32) / TEMP
    sB = lax.dot_general(anchorQ, aemb, (((1,), (1,)), ((), ())),
                         preferred_element_type=jnp.float32) / TEMP
    sFm = jnp.where(maskA, sF, NEG)
    sBm = jnp.where(maskB, sB, NEG)
    has_pad = lenQ < maxQ
    m = jnp.maximum(jnp.max(sFm, axis=1, keepdims=True),
                    jnp.max(sBm, axis=1, keepdims=True))
    m = jnp.maximum(m, jnp.where(has_pad, 0.0, NEG))
    expF = jnp.where(maskA, jnp.exp(sFm - m), 0.0)
    expB = jnp.where(maskB, jnp.exp(sBm - m), 0.0)
    logsum = jnp.log(jnp.sum(expF, axis=1, keepdims=True)
                     + jnp.sum(expB, axis=1, keepdims=True)
                     + (maxQ - lenQ) * jnp.exp(-m))
    numsum = (jnp.sum(jnp.where(posmask, num, 0.0), axis=1, keepdims=True)
              + (maxP - lenP) * pad_num)
    mean_lp = (numsum - maxP * logsum) / maxP
    loss = -(TEMP / BASE_TEMP) * mean_lp
    total = jnp.sum(jnp.where(cond_col, loss, 0.0)) / n_sel
    out_ref[...] = jnp.full((1, 128), total, jnp.float32)


def kernel(label, samples_of_further_pairs, class_of_further_pair,
           idx_further_pair, samples_of_closest_pairs, class_of_closest_pair,
           idx_closest_pair, W1, b1, W2, b2):
    Xc = samples_of_closest_pairs.reshape(128, FLAT)
    Xf = samples_of_further_pairs.reshape(256, FLAT)
    labf = label.astype(jnp.float32)
    lab_row = labf.reshape(1, 512)
    lab_col = labf.reshape(512, 1)
    ic = idx_closest_pair.astype(jnp.int32)
    ic_row = ic.reshape(1, 128)
    ic_col = ic.reshape(128, 1)
    iff_row = idx_further_pair.astype(jnp.int32).reshape(1, 256)
    ccpf = class_of_closest_pair.astype(jnp.float32)
    t1_row = ccpf[:, 0].reshape(1, 128)
    t2_row = ccpf[:, 1].reshape(1, 128)
    t1_col = ccpf[:, 0].reshape(128, 1)
    t2_col = ccpf[:, 1].reshape(128, 1)
    clsf_row = class_of_further_pair[:, 0].astype(jnp.float32).reshape(1, 256)
    b1r = b1.reshape(1, 128)
    b2r = b2.reshape(1, 128)

    pc, pf = pl.pallas_call(
        _mm_kernel,
        grid=(PAR, HALF),
        in_specs=[
            pl.BlockSpec((128, KT), lambda m, k: (0, m * HALF + k)),
            pl.BlockSpec((256, KT), lambda m, k: (0, m * HALF + k)),
            pl.BlockSpec((KT, 128), lambda m, k: (m * HALF + k, 0)),
        ],
        out_specs=[
            pl.BlockSpec((1, 128, 128), lambda m, k: (m, 0, 0)),
            pl.BlockSpec((1, 256, 128), lambda m, k: (m, 0, 0)),
        ],
        out_shape=[jax.ShapeDtypeStruct((PAR, 128, 128), jnp.float32),
                   jax.ShapeDtypeStruct((PAR, 256, 128), jnp.float32)],
        scratch_shapes=[pltpu.VMEM((128, 128), jnp.float32),
                        pltpu.VMEM((256, 128), jnp.float32)],
        compiler_params=pltpu.CompilerParams(
            dimension_semantics=("parallel", "arbitrary")),
        interpret=_INTERPRET,
    )(Xc, Xf, W1)

    def full(shape):
        return pl.BlockSpec(shape, lambda: (0,) * len(shape))

    out = pl.pallas_call(
        _loss_kernel,
        in_specs=[
            full((PAR, 128, 128)), full((PAR, 256, 128)),
            full((1, 512)), full((512, 1)), full((1, 128)), full((128, 1)),
            full((1, 256)), full((1, 128)), full((1, 128)), full((128, 1)),
            full((128, 1)), full((1, 256)), full((1, 128)), full((1, 128)),
            full((128, 128)),
        ],
        out_specs=full((1, 128)),
        out_shape=jax.ShapeDtypeStruct((1, 128), jnp.float32),
        interpret=_INTERPRET,
    )(pc, pf, lab_row, lab_col, ic_row, ic_col, iff_row,
      t1_row, t2_row, t1_col, t2_col, clsf_row, b1r, b2r, W2)
    return out[0, 0]
